# Initial kernel scaffold; baseline (speedup 1.0000x reference)
import jax
import jax.numpy as jnp
from jax.experimental import pallas as pl


def kernel(x, embeds):
    raise NotImplementedError("write your pallas kernel here")


# speedup vs baseline: 4.0895x; 4.0895x over previous
import functools

import jax
import jax.numpy as jnp
from jax import lax
from jax.experimental import pallas as pl
from jax.experimental.pallas import tpu as pltpu, tpu_sc as plsc

EMBED_N = 8192
EMBED_D = 32
TM = 256


def _argmin_body(f_ref, e_ref, idx_ref, ms_ref):
    f = f_ref[...]
    e = e_ref[...]
    sim = jnp.dot(f, e, preferred_element_type=jnp.float32)
    f2 = jnp.sum(f * f, axis=1, keepdims=True)
    e2 = jnp.sum(e * e, axis=0, keepdims=True)
    dists = (f2 + e2) - 2.0 * sim
    minv = jnp.min(dists, axis=1, keepdims=True)
    iota = lax.broadcasted_iota(jnp.int32, dists.shape, 1)
    cand = jnp.where(dists == minv, iota, EMBED_N)
    idx = jnp.min(cand, axis=1).astype(jnp.int32)
    idx_ref[0, 0, :] = idx
    i = pl.program_id(0)
    prev = jnp.where(i == 0, jnp.zeros((1, 1), jnp.float32), ms_ref[...])
    ms_ref[...] = prev + jnp.sum(minv)


def _compute_indices(flat, embeds):
    grid = flat.shape[0] // TM
    idx3, minsum = pl.pallas_call(
        _argmin_body,
        grid=(grid,),
        in_specs=[
            pl.BlockSpec((TM, EMBED_D), lambda i: (i, 0)),
            pl.BlockSpec((EMBED_D, EMBED_N), lambda i: (0, 0)),
        ],
        out_specs=[
            pl.BlockSpec((1, 1, TM), lambda i: (i, 0, 0)),
            pl.BlockSpec((1, 1), lambda i: (0, 0)),
        ],
        out_shape=[
            jax.ShapeDtypeStruct((grid, 1, TM), jnp.int32),
            jax.ShapeDtypeStruct((1, 1), jnp.float32),
        ],
    )(flat, embeds)
    return idx3.reshape(-1), minsum[0, 0]


def _make_sc_gather(B, D):
    info = plsc.get_sparse_core_info()
    NC, NS, L = info.num_cores, info.num_subcores, info.num_lanes
    NW = NC * NS
    b_per_w = B // NW
    CHUNK = 128
    n_chunks = b_per_w // CHUNK
    mesh = plsc.VectorSubcoreMesh(core_axis_name="c", subcore_axis_name="s")

    @functools.partial(
        pl.kernel,
        mesh=mesh,
        compiler_params=pltpu.CompilerParams(use_tc_tiling_on_sc=False),
        out_type=jax.ShapeDtypeStruct((B, D), jnp.float32),
        scratch_types=[
            pltpu.VMEM((n_chunks, CHUNK), jnp.int32),
            pltpu.VMEM((b_per_w, D), jnp.float32),
            pltpu.SemaphoreType.DMA,
        ],
    )
    def gather_k(table_hbm, idx_hbm, out_hbm, idx_v, rows_v, sem):
        wid = lax.axis_index("s") * NC + lax.axis_index("c")
        base = wid * b_per_w
        pltpu.sync_copy(idx_hbm.at[wid], idx_v)
        cps = [
            pltpu.async_copy(
                table_hbm.at[idx_v.at[j]],
                rows_v.at[pl.ds(j * CHUNK, CHUNK)],
                sem,
            )
            for j in range(n_chunks)
        ]
        for c in cps:
            c.wait()
        pltpu.sync_copy(rows_v, out_hbm.at[pl.ds(base, b_per_w)])

    def run(table, idx):
        idx_r = idx.reshape(NW, n_chunks, CHUNK)
        return gather_k(table, idx_r)

    return run


_sc_gather = None


def kernel(x, embeds):
    global _sc_gather
    shape = x.shape
    flat = x.reshape(-1, EMBED_D)
    B = flat.shape[0]
    idx, minsum = _compute_indices(flat, embeds)
    if _sc_gather is None:
        _sc_gather = _make_sc_gather(B, EMBED_D)
    table = embeds.T
    quantized = _sc_gather(table, idx).reshape(shape)
    loss = (1.0 + 0.25) * minsum / (B * EMBED_D)
    return quantized, loss


# speedup vs baseline: 4.5860x; 1.1214x over previous
import functools

import jax
import jax.numpy as jnp
from jax import lax
from jax.experimental import pallas as pl
from jax.experimental.pallas import tpu as pltpu, tpu_sc as plsc

EMBED_N = 8192
EMBED_D = 32
TM = 256


CH = 512


def _argmin_body(f_ref, e_ref, idx_ref, ms_ref):
    f = f_ref[...]
    e = e_ref[...]
    sim2 = jnp.dot(f + f, e, preferred_element_type=jnp.float32)
    f2 = jnp.sum(f * f, axis=1, keepdims=True)
    e2 = jnp.sum(e * e, axis=0, keepdims=True)
    n_chunks = EMBED_N // CH
    rm = (f2 + e2[:, 0:CH]) - sim2[:, 0:CH]
    ri = jnp.zeros(rm.shape, jnp.int32)
    for c in range(1, n_chunks):
        d = (f2 + e2[:, c * CH:(c + 1) * CH]) - sim2[:, c * CH:(c + 1) * CH]
        m = d < rm
        rm = jnp.minimum(rm, d)
        ri = jnp.where(m, c, ri)
    minv = jnp.min(rm, axis=1, keepdims=True)
    lane = lax.broadcasted_iota(jnp.int32, rm.shape, 1)
    cand = jnp.where(rm == minv, ri * CH + lane, EMBED_N)
    idx = jnp.min(cand, axis=1).astype(jnp.int32)
    idx_ref[0, 0, :] = idx
    i = pl.program_id(0)
    prev = jnp.where(i == 0, jnp.zeros((1, 1), jnp.float32), ms_ref[...])
    ms_ref[...] = prev + jnp.sum(minv)


def _compute_indices(flat, embeds):
    grid = flat.shape[0] // TM
    idx3, minsum = pl.pallas_call(
        _argmin_body,
        grid=(grid,),
        in_specs=[
            pl.BlockSpec((TM, EMBED_D), lambda i: (i, 0)),
            pl.BlockSpec((EMBED_D, EMBED_N), lambda i: (0, 0)),
        ],
        out_specs=[
            pl.BlockSpec((1, 1, TM), lambda i: (i, 0, 0)),
            pl.BlockSpec((1, 1), lambda i: (0, 0)),
        ],
        out_shape=[
            jax.ShapeDtypeStruct((grid, 1, TM), jnp.int32),
            jax.ShapeDtypeStruct((1, 1), jnp.float32),
        ],
    )(flat, embeds)
    return idx3.reshape(-1), minsum[0, 0]


def _make_sc_gather(B, D):
    info = plsc.get_sparse_core_info()
    NC, NS, L = info.num_cores, info.num_subcores, info.num_lanes
    NW = NC * NS
    b_per_w = B // NW
    CHUNK = 128
    n_chunks = b_per_w // CHUNK
    mesh = plsc.VectorSubcoreMesh(core_axis_name="c", subcore_axis_name="s")

    @functools.partial(
        pl.kernel,
        mesh=mesh,
        compiler_params=pltpu.CompilerParams(use_tc_tiling_on_sc=False),
        out_type=jax.ShapeDtypeStruct((B, D), jnp.float32),
        scratch_types=[
            pltpu.VMEM((n_chunks, CHUNK), jnp.int32),
            pltpu.VMEM((b_per_w, D), jnp.float32),
            pltpu.SemaphoreType.DMA,
        ],
    )
    def gather_k(table_hbm, idx_hbm, out_hbm, idx_v, rows_v, sem):
        wid = lax.axis_index("s") * NC + lax.axis_index("c")
        base = wid * b_per_w
        pltpu.sync_copy(idx_hbm.at[wid], idx_v)
        cps = [
            pltpu.async_copy(
                table_hbm.at[idx_v.at[j]],
                rows_v.at[pl.ds(j * CHUNK, CHUNK)],
                sem,
            )
            for j in range(n_chunks)
        ]
        for c in cps:
            c.wait()
        pltpu.sync_copy(rows_v, out_hbm.at[pl.ds(base, b_per_w)])

    def run(table, idx):
        idx_r = idx.reshape(NW, n_chunks, CHUNK)
        return gather_k(table, idx_r)

    return run


_sc_gather = None


def kernel(x, embeds):
    global _sc_gather
    shape = x.shape
    flat = x.reshape(-1, EMBED_D)
    B = flat.shape[0]
    idx, minsum = _compute_indices(flat, embeds)
    if _sc_gather is None:
        _sc_gather = _make_sc_gather(B, EMBED_D)
    table = embeds.T
    quantized = _sc_gather(table, idx).reshape(shape)
    loss = (1.0 + 0.25) * minsum / (B * EMBED_D)
    return quantized, loss


# speedup vs baseline: 4.8088x; 1.0486x over previous
import functools

import jax
import jax.numpy as jnp
from jax import lax
from jax.experimental import pallas as pl
from jax.experimental.pallas import tpu as pltpu, tpu_sc as plsc

EMBED_N = 8192
EMBED_D = 32
TM = 256


CH = 128


def _argmin_body(f_ref, e_ref, idx_ref, ms_ref, et_ref):
    f = f_ref[...]
    e = e_ref[...]
    sim2 = jnp.dot(f + f, e, preferred_element_type=jnp.float32)
    f2 = jnp.sum(f * f, axis=1, keepdims=True)
    e2 = jnp.sum(e * e, axis=0, keepdims=True)
    n_chunks = EMBED_N // CH
    rm = (f2 + e2[:, 0:CH]) - sim2[:, 0:CH]
    ri = jnp.zeros(rm.shape, jnp.int32)
    for c in range(1, n_chunks):
        d = (f2 + e2[:, c * CH:(c + 1) * CH]) - sim2[:, c * CH:(c + 1) * CH]
        m = d < rm
        rm = jnp.minimum(rm, d)
        ri = jnp.where(m, c, ri)
    minv = jnp.min(rm, axis=1, keepdims=True)
    lane = lax.broadcasted_iota(jnp.int32, rm.shape, 1)
    cand = jnp.where(rm == minv, ri * CH + lane, EMBED_N)
    idx = jnp.min(cand, axis=1).astype(jnp.int32)
    idx_ref[0, 0, :] = idx
    i = pl.program_id(0)
    prev = jnp.where(i == 0, jnp.zeros((1, 1), jnp.float32), ms_ref[...])
    ms_ref[...] = prev + jnp.sum(minv)

    @pl.when(i == 0)
    def _():
        et_ref[...] = jnp.swapaxes(e, 0, 1)


def _compute_indices(flat, embeds):
    grid = flat.shape[0] // TM
    idx3, minsum, et = pl.pallas_call(
        _argmin_body,
        grid=(grid,),
        in_specs=[
            pl.BlockSpec((TM, EMBED_D), lambda i: (i, 0)),
            pl.BlockSpec((EMBED_D, EMBED_N), lambda i: (0, 0)),
        ],
        out_specs=[
            pl.BlockSpec((1, 1, TM), lambda i: (i, 0, 0)),
            pl.BlockSpec((1, 1), lambda i: (0, 0)),
            pl.BlockSpec((EMBED_N, EMBED_D), lambda i: (0, 0)),
        ],
        out_shape=[
            jax.ShapeDtypeStruct((grid, 1, TM), jnp.int32),
            jax.ShapeDtypeStruct((1, 1), jnp.float32),
            jax.ShapeDtypeStruct((EMBED_N, EMBED_D), jnp.float32),
        ],
    )(flat, embeds)
    return idx3.reshape(-1), minsum[0, 0], et


def _make_sc_gather(B, D):
    info = plsc.get_sparse_core_info()
    NC, NS, L = info.num_cores, info.num_subcores, info.num_lanes
    NW = NC * NS
    b_per_w = B // NW
    CHUNK = 128
    n_chunks = b_per_w // CHUNK
    mesh = plsc.VectorSubcoreMesh(core_axis_name="c", subcore_axis_name="s")

    @functools.partial(
        pl.kernel,
        mesh=mesh,
        compiler_params=pltpu.CompilerParams(use_tc_tiling_on_sc=False),
        out_type=jax.ShapeDtypeStruct((B, D), jnp.float32),
        scratch_types=[
            pltpu.VMEM((n_chunks, CHUNK), jnp.int32),
            pltpu.VMEM((b_per_w, D), jnp.float32),
            pltpu.SemaphoreType.DMA,
        ],
    )
    def gather_k(table_hbm, idx_hbm, out_hbm, idx_v, rows_v, sem):
        wid = lax.axis_index("s") * NC + lax.axis_index("c")
        base = wid * b_per_w
        pltpu.sync_copy(idx_hbm.at[wid], idx_v)
        cps = [
            pltpu.async_copy(
                table_hbm.at[idx_v.at[j]],
                rows_v.at[pl.ds(j * CHUNK, CHUNK)],
                sem,
            )
            for j in range(n_chunks)
        ]
        for c in cps:
            c.wait()
        pltpu.sync_copy(rows_v, out_hbm.at[pl.ds(base, b_per_w)])

    def run(table, idx):
        idx_r = idx.reshape(NW, n_chunks, CHUNK)
        return gather_k(table, idx_r)

    return run


_sc_gather = None


def kernel(x, embeds):
    global _sc_gather
    shape = x.shape
    flat = x.reshape(-1, EMBED_D)
    B = flat.shape[0]
    idx, minsum, table = _compute_indices(flat, embeds)
    if _sc_gather is None:
        _sc_gather = _make_sc_gather(B, EMBED_D)
    quantized = _sc_gather(table, idx).reshape(shape)
    loss = (1.0 + 0.25) * minsum / (B * EMBED_D)
    return quantized, loss


# speedup vs baseline: 5.5597x; 1.1561x over previous
import functools

import jax
import jax.numpy as jnp
from jax import lax
from jax.experimental import pallas as pl
from jax.experimental.pallas import tpu as pltpu, tpu_sc as plsc

EMBED_N = 8192
EMBED_D = 32
TM = 512
CH = 128
BETA = 0.25


def _argmin_body(f_ref, e_ref, idx_ref, ms_ref, et_ref):
    f = f_ref[0]
    e = e_ref[...]
    sim2 = jnp.dot(f + f, e, preferred_element_type=jnp.float32)
    f2 = jnp.sum(f * f, axis=1, keepdims=True)
    e2 = jnp.sum(e * e, axis=0, keepdims=True)
    n_chunks = EMBED_N // CH
    rm = (f2 + e2[:, 0:CH]) - sim2[:, 0:CH]
    ri = jnp.zeros(rm.shape, jnp.int32)
    for c in range(1, n_chunks):
        d = (f2 + e2[:, c * CH:(c + 1) * CH]) - sim2[:, c * CH:(c + 1) * CH]
        m = d < rm
        rm = jnp.minimum(rm, d)
        ri = jnp.where(m, c, ri)
    minv = jnp.min(rm, axis=1, keepdims=True)
    lane = lax.broadcasted_iota(jnp.int32, rm.shape, 1)
    cand = jnp.where(rm == minv, ri * CH + lane, EMBED_N)
    idx = jnp.min(cand, axis=1).astype(jnp.int32)
    idx_ref[...] = idx.reshape(1, TM // CH, CH)
    i = pl.program_id(0)
    n = pl.num_programs(0)
    prev = jnp.where(i == 0, jnp.zeros((1, 1), jnp.float32), ms_ref[...])
    total = prev + jnp.sum(minv)
    scale = (1.0 + BETA) / (n * TM * EMBED_D)
    ms_ref[...] = jnp.where(i == n - 1, total * scale, total)

    @pl.when(i == 0)
    def _():
        et_ref[...] = jnp.swapaxes(e, 0, 1)


def _compute_indices(x, embeds):
    B = x.shape[0] * x.shape[1]
    grid = B // TM
    per_b = x.shape[1] // TM
    idx3, loss_arr, et = pl.pallas_call(
        _argmin_body,
        grid=(grid,),
        in_specs=[
            pl.BlockSpec((1, TM, EMBED_D), lambda i: (i // per_b, i % per_b, 0)),
            pl.BlockSpec((EMBED_D, EMBED_N), lambda i: (0, 0)),
        ],
        out_specs=[
            pl.BlockSpec((1, TM // CH, CH), lambda i: (i, 0, 0)),
            pl.BlockSpec((1, 1), lambda i: (0, 0)),
            pl.BlockSpec((EMBED_N, EMBED_D), lambda i: (0, 0)),
        ],
        out_shape=[
            jax.ShapeDtypeStruct((grid, TM // CH, CH), jnp.int32),
            jax.ShapeDtypeStruct((1, 1), jnp.float32),
            jax.ShapeDtypeStruct((EMBED_N, EMBED_D), jnp.float32),
        ],
    )(x, embeds)
    return idx3, loss_arr, et


def _make_sc_gather(out_shape):
    info = plsc.get_sparse_core_info()
    NC, NS, L = info.num_cores, info.num_subcores, info.num_lanes
    NW = NC * NS
    BATCH, SEQ, D = out_shape
    b_per_w = BATCH * SEQ // NW
    seq_per_w = SEQ // b_per_w
    CHUNK = 128
    n_chunks = b_per_w // CHUNK
    mesh = plsc.VectorSubcoreMesh(core_axis_name="c", subcore_axis_name="s")

    @functools.partial(
        pl.kernel,
        mesh=mesh,
        compiler_params=pltpu.CompilerParams(use_tc_tiling_on_sc=False),
        out_type=jax.ShapeDtypeStruct(out_shape, jnp.float32),
        scratch_types=[
            pltpu.VMEM((n_chunks, CHUNK), jnp.int32),
            pltpu.VMEM((b_per_w, D), jnp.float32),
            pltpu.SemaphoreType.DMA,
        ],
    )
    def gather_k(table_hbm, idx_hbm, out_hbm, idx_v, rows_v, sem):
        wid = lax.axis_index("s") * NC + lax.axis_index("c")
        b = wid // seq_per_w
        off = (wid % seq_per_w) * b_per_w
        pltpu.sync_copy(idx_hbm.at[wid], idx_v)
        cps = [
            pltpu.async_copy(
                table_hbm.at[idx_v.at[j]],
                rows_v.at[pl.ds(j * CHUNK, CHUNK)],
                sem,
            )
            for j in range(n_chunks)
        ]
        for c in cps:
            c.wait()
        pltpu.sync_copy(rows_v, out_hbm.at[b, pl.ds(off, b_per_w)])

    return gather_k


_sc_gather = None


def kernel(x, embeds):
    global _sc_gather
    idx3, loss_arr, table = _compute_indices(x, embeds)
    if _sc_gather is None:
        _sc_gather = _make_sc_gather(x.shape)
    quantized = _sc_gather(table, idx3)
    return quantized, loss_arr[0, 0]


# speedup vs baseline: 5.5818x; 1.0040x over previous
import functools

import jax
import jax.numpy as jnp
from jax import lax
from jax.experimental import pallas as pl
from jax.experimental.pallas import tpu as pltpu, tpu_sc as plsc

EMBED_N = 8192
EMBED_D = 32
TM = 512
CH = 128
BETA = 0.25


def _transpose_body(e_ref, et_ref):
    et_ref[...] = jnp.swapaxes(e_ref[...], 0, 1)


def _transpose_table(embeds):
    return pl.pallas_call(
        _transpose_body,
        out_shape=jax.ShapeDtypeStruct((EMBED_N, EMBED_D), jnp.float32),
    )(embeds)


def _argmin_body(f_ref, e_ref, idx_ref, ms_ref):
    f = f_ref[0]
    e = e_ref[...]
    sim2 = jnp.dot(f + f, e, preferred_element_type=jnp.float32)
    f2 = jnp.sum(f * f, axis=1, keepdims=True)
    e2 = jnp.sum(e * e, axis=0, keepdims=True)
    n_chunks = EMBED_N // CH
    rm = (f2 + e2[:, 0:CH]) - sim2[:, 0:CH]
    ri = jnp.zeros(rm.shape, jnp.int32)
    for c in range(1, n_chunks):
        d = (f2 + e2[:, c * CH:(c + 1) * CH]) - sim2[:, c * CH:(c + 1) * CH]
        m = d < rm
        rm = jnp.minimum(rm, d)
        ri = jnp.where(m, c, ri)
    minv = jnp.min(rm, axis=1, keepdims=True)
    lane = lax.broadcasted_iota(jnp.int32, rm.shape, 1)
    cand = jnp.where(rm == minv, ri * CH + lane, EMBED_N)
    idx = jnp.min(cand, axis=1).astype(jnp.int32)
    idx_ref[...] = idx.reshape(1, TM // 128, 128)
    i = pl.program_id(0)
    n = pl.num_programs(0)
    prev = jnp.where(i == 0, jnp.zeros((1, 1), jnp.float32), ms_ref[...])
    total = prev + jnp.sum(minv)
    scale = (1.0 + BETA) / (n * TM * EMBED_D)
    ms_ref[...] = jnp.where(i == n - 1, total * scale, total)


def _compute_indices(x, embeds):
    B = x.shape[0] * x.shape[1]
    grid = B // TM
    per_b = x.shape[1] // TM
    idx3, loss_arr = pl.pallas_call(
        _argmin_body,
        grid=(grid,),
        in_specs=[
            pl.BlockSpec((1, TM, EMBED_D), lambda i: (i // per_b, i % per_b, 0)),
            pl.BlockSpec((EMBED_D, EMBED_N), lambda i: (0, 0)),
        ],
        out_specs=[
            pl.BlockSpec((1, TM // 128, 128), lambda i: (i, 0, 0)),
            pl.BlockSpec((1, 1), lambda i: (0, 0)),
        ],
        out_shape=[
            jax.ShapeDtypeStruct((grid, TM // 128, 128), jnp.int32),
            jax.ShapeDtypeStruct((1, 1), jnp.float32),
        ],
    )(x, embeds)
    return idx3, loss_arr


def _make_sc_gather(out_shape):
    info = plsc.get_sparse_core_info()
    NC, NS, L = info.num_cores, info.num_subcores, info.num_lanes
    NW = NC * NS
    BATCH, SEQ, D = out_shape
    b_per_w = BATCH * SEQ // NW
    seq_per_w = SEQ // b_per_w
    CHUNK = 128
    n_chunks = b_per_w // CHUNK
    mesh = plsc.VectorSubcoreMesh(core_axis_name="c", subcore_axis_name="s")

    @functools.partial(
        pl.kernel,
        mesh=mesh,
        compiler_params=pltpu.CompilerParams(use_tc_tiling_on_sc=False),
        out_type=jax.ShapeDtypeStruct(out_shape, jnp.float32),
        scratch_types=[
            pltpu.VMEM((n_chunks, CHUNK), jnp.int32),
            pltpu.VMEM((b_per_w, D), jnp.float32),
            pltpu.SemaphoreType.DMA,
        ],
    )
    def gather_k(table_hbm, idx_hbm, out_hbm, idx_v, rows_v, sem):
        wid = lax.axis_index("s") * NC + lax.axis_index("c")
        b = wid // seq_per_w
        off = (wid % seq_per_w) * b_per_w
        pltpu.sync_copy(idx_hbm.at[wid], idx_v)
        cps = [
            pltpu.async_copy(
                table_hbm.at[idx_v.at[j]],
                rows_v.at[pl.ds(j * CHUNK, CHUNK)],
                sem,
            )
            for j in range(n_chunks)
        ]
        for c in cps:
            c.wait()
        pltpu.sync_copy(rows_v, out_hbm.at[b, pl.ds(off, b_per_w)])

    return gather_k


_sc_gather = None


def kernel(x, embeds):
    global _sc_gather
    table = _transpose_table(embeds)
    idx3, loss_arr = _compute_indices(x, embeds)
    if _sc_gather is None:
        _sc_gather = _make_sc_gather(x.shape)
    quantized = _sc_gather(table, idx3)
    return quantized, loss_arr[0, 0]


# speedup vs baseline: 5.8931x; 1.0558x over previous
import functools

import jax
import jax.numpy as jnp
from jax import lax
from jax.experimental import pallas as pl
from jax.experimental.pallas import tpu as pltpu, tpu_sc as plsc

EMBED_N = 8192
EMBED_D = 32
TM = 1024
CH = 128
BETA = 0.25


def _transpose_body(e_ref, et_ref):
    et_ref[...] = jnp.swapaxes(e_ref[...], 0, 1)


def _transpose_table(embeds):
    return pl.pallas_call(
        _transpose_body,
        out_shape=jax.ShapeDtypeStruct((EMBED_N, EMBED_D), jnp.float32),
    )(embeds)


def _argmin_body(f_ref, e_ref, idx_ref, ms_ref):
    f = f_ref[0]
    e = e_ref[...]
    sim2 = jnp.dot(f + f, e, preferred_element_type=jnp.float32)
    f2 = jnp.sum(f * f, axis=1, keepdims=True)
    e2 = jnp.sum(e * e, axis=0, keepdims=True)
    n_chunks = EMBED_N // CH
    rm = (f2 + e2[:, 0:CH]) - sim2[:, 0:CH]
    ri = jnp.zeros(rm.shape, jnp.int32)
    for c in range(1, n_chunks):
        d = (f2 + e2[:, c * CH:(c + 1) * CH]) - sim2[:, c * CH:(c + 1) * CH]
        m = d < rm
        rm = jnp.minimum(rm, d)
        ri = jnp.where(m, c, ri)
    minv = jnp.min(rm, axis=1, keepdims=True)
    lane = lax.broadcasted_iota(jnp.int32, rm.shape, 1)
    cand = jnp.where(rm == minv, ri * CH + lane, EMBED_N)
    idx = jnp.min(cand, axis=1).astype(jnp.int32)
    idx_ref[...] = idx.reshape(TM // 512, 4, 128)
    i = pl.program_id(0)
    n = pl.num_programs(0)
    prev = jnp.where(i == 0, jnp.zeros((1, 1), jnp.float32), ms_ref[...])
    total = prev + jnp.sum(minv)
    scale = (1.0 + BETA) / (n * TM * EMBED_D)
    ms_ref[...] = jnp.where(i == n - 1, total * scale, total)


def _compute_indices(x, embeds):
    B = x.shape[0] * x.shape[1]
    grid = B // TM
    per_b = x.shape[1] // TM
    idx3, loss_arr = pl.pallas_call(
        _argmin_body,
        grid=(grid,),
        in_specs=[
            pl.BlockSpec((1, TM, EMBED_D), lambda i: (i // per_b, i % per_b, 0)),
            pl.BlockSpec((EMBED_D, EMBED_N), lambda i: (0, 0)),
        ],
        out_specs=[
            pl.BlockSpec((TM // 512, 4, 128), lambda i: (i, 0, 0)),
            pl.BlockSpec((1, 1), lambda i: (0, 0)),
        ],
        out_shape=[
            jax.ShapeDtypeStruct((B // 512, 4, 128), jnp.int32),
            jax.ShapeDtypeStruct((1, 1), jnp.float32),
        ],
    )(x, embeds)
    return idx3, loss_arr


def _make_sc_gather(out_shape):
    info = plsc.get_sparse_core_info()
    NC, NS, L = info.num_cores, info.num_subcores, info.num_lanes
    NW = NC * NS
    BATCH, SEQ, D = out_shape
    b_per_w = BATCH * SEQ // NW
    seq_per_w = SEQ // b_per_w
    CHUNK = 128
    n_chunks = b_per_w // CHUNK
    mesh = plsc.VectorSubcoreMesh(core_axis_name="c", subcore_axis_name="s")

    @functools.partial(
        pl.kernel,
        mesh=mesh,
        compiler_params=pltpu.CompilerParams(use_tc_tiling_on_sc=False),
        out_type=jax.ShapeDtypeStruct(out_shape, jnp.float32),
        scratch_types=[
            pltpu.VMEM((n_chunks, CHUNK), jnp.int32),
            pltpu.VMEM((b_per_w, D), jnp.float32),
            pltpu.SemaphoreType.DMA,
        ],
    )
    def gather_k(table_hbm, idx_hbm, out_hbm, idx_v, rows_v, sem):
        wid = lax.axis_index("s") * NC + lax.axis_index("c")
        b = wid // seq_per_w
        off = (wid % seq_per_w) * b_per_w
        pltpu.sync_copy(idx_hbm.at[wid], idx_v)
        cps = [
            pltpu.async_copy(
                table_hbm.at[idx_v.at[j]],
                rows_v.at[pl.ds(j * CHUNK, CHUNK)],
                sem,
            )
            for j in range(n_chunks)
        ]
        for c in cps:
            c.wait()
        pltpu.sync_copy(rows_v, out_hbm.at[b, pl.ds(off, b_per_w)])

    return gather_k


_sc_gather = None


def kernel(x, embeds):
    global _sc_gather
    table = _transpose_table(embeds)
    idx3, loss_arr = _compute_indices(x, embeds)
    if _sc_gather is None:
        _sc_gather = _make_sc_gather(x.shape)
    quantized = _sc_gather(table, idx3)
    return quantized, loss_arr[0, 0]
